# Initial kernel scaffold; baseline (speedup 1.0000x reference)
import jax
import jax.numpy as jnp
from jax.experimental import pallas as pl


def kernel(edge_index, emb, W_hg, b_hg, W_lin, b_lin):
    raise NotImplementedError("write your pallas kernel here")


# speedup vs baseline: 6.2361x; 6.2361x over previous
import dataclasses
import functools

import jax
import jax.numpy as jnp
from jax import lax
from jax.experimental import pallas as pl
from jax.experimental.pallas import tpu as pltpu
from jax.experimental.pallas import tpu_sc as plsc

N_NODES = 10000
NNZ = 320000
C = 128

NC = 2
NS = 16
NW = NC * NS
PAD_ROWS = 240
R = N_NODES + PAD_ROWS
RPT = R // NW
NNZ_PAD = 327680
CHUNK = 8192
NCHUNK = NNZ_PAD // CHUNK
CAP = 12288
GB = 128
TCB = 2560
TCG = R // TCB

_MESH = plsc.VectorSubcoreMesh(core_axis_name="c", subcore_axis_name="s")
_CP = pltpu.CompilerParams()
if "needs_layout_passes" in pltpu.CompilerParams.__dataclass_fields__:
    _CP = dataclasses.replace(_CP, needs_layout_passes=False)


def _mm1_body(emb_ref, w_ref, o_ref):
    o_ref[...] = lax.dot_general(
        emb_ref[...], w_ref[...], (((1,), (1,)), ((), ())),
        preferred_element_type=jnp.float32)


def _mm1(emb_p, w_hg):
    return pl.pallas_call(
        _mm1_body,
        grid=(TCG,),
        in_specs=[pl.BlockSpec((TCB, C), lambda i: (i, 0)),
                  pl.BlockSpec((C, C), lambda i: (0, 0))],
        out_specs=pl.BlockSpec((TCB, C), lambda i: (i, 0)),
        out_shape=jax.ShapeDtypeStruct((R, C), jnp.float32),
    )(emb_p, w_hg)


def _final_body(p_ref, bhg_ref, wlin_ref, blin_ref, o_ref):
    s = p_ref[...] + bhg_ref[...]
    t = jnp.where(s >= 0.0, s, 0.01 * s)
    o_ref[...] = lax.dot_general(
        t, wlin_ref[...], (((1,), (1,)), ((), ())),
        preferred_element_type=jnp.float32) + blin_ref[...]


def _final(out_sc, b_hg, w_lin, b_lin):
    return pl.pallas_call(
        _final_body,
        grid=(TCG,),
        in_specs=[pl.BlockSpec((TCB, C), lambda i: (i, 0)),
                  pl.BlockSpec((1, C), lambda i: (0, 0)),
                  pl.BlockSpec((C, C), lambda i: (0, 0)),
                  pl.BlockSpec((1, C), lambda i: (0, 0))],
        out_specs=pl.BlockSpec((TCB, C), lambda i: (i, 0)),
        out_shape=jax.ShapeDtypeStruct((N_NODES, C), jnp.float32),
    )(out_sc, b_hg, w_lin, b_lin)


def _segment_mean(table, dst_flat, src_flat):
    @functools.partial(
        pl.kernel,
        out_type=jax.ShapeDtypeStruct((R, C), jnp.float32),
        mesh=_MESH,
        compiler_params=_CP,
        scratch_types=[
            pltpu.VMEM((CHUNK,), jnp.int32),
            pltpu.VMEM((CHUNK,), jnp.int32),
            pltpu.VMEM((CAP,), jnp.int32),
            pltpu.VMEM((CAP,), jnp.int32),
            pltpu.VMEM((RPT,), jnp.float32),
            pltpu.VMEM((GB, C), jnp.float32),
            pltpu.VMEM((RPT, C), jnp.float32),
        ],
    )
    def k(tab_hbm, dst_hbm, src_hbm, out_hbm,
          dch_v, sch_v, pdst_v, psrc_v, hist_v, rows_v, acc_v):
        cid = lax.axis_index("c")
        sid = lax.axis_index("s")
        wid = sid * NC + cid
        lo = wid * RPT

        zero16 = jnp.zeros(16, jnp.float32)
        one16 = jnp.ones(16, jnp.float32)

        @pl.loop(0, RPT)
        def _(r):
            for c in range(8):
                acc_v[r, pl.ds(c * 16, 16)] = zero16

        @pl.loop(0, RPT, step=16)
        def _(i):
            hist_v[pl.ds(i, 16)] = zero16

        @pl.loop(0, CAP, step=16)
        def _(i):
            psrc_v[pl.ds(i, 16)] = jnp.full(16, N_NODES, jnp.int32)
            pdst_v[pl.ds(i, 16)] = jnp.zeros(16, jnp.int32)

        def chunk_body(ch, cnt):
            pltpu.sync_copy(dst_hbm.at[pl.ds(ch * CHUNK, CHUNK)], dch_v)
            pltpu.sync_copy(src_hbm.at[pl.ds(ch * CHUNK, CHUNK)], sch_v)

            def vec_body(i, cnt):
                dv = dch_v[pl.ds(i * 16, 16)]
                sv = sch_v[pl.ds(i * 16, 16)]
                dl = dv - lo
                own = (dl >= 0) & (dl < RPT)
                oi = jnp.where(own, 1, 0)
                pre = plsc.cumsum(oi)
                pos = jnp.minimum(cnt + pre - oi, CAP - 1)
                plsc.store_scatter(psrc_v, [pos], sv, mask=own)
                plsc.store_scatter(pdst_v, [pos], dl, mask=own)
                plsc.addupdate_scatter(hist_v, [dl], one16, mask=own)
                return jnp.minimum(cnt + pre[15], jnp.int32(CAP))

            return lax.fori_loop(0, CHUNK // 16, vec_body, cnt)

        cnt = lax.fori_loop(0, NCHUNK, chunk_body, jnp.int32(0))

        nb = (cnt + GB - 1) // GB

        def batch_body(b, carry):
            pltpu.sync_copy(tab_hbm.at[psrc_v.at[pl.ds(b * GB, GB)]], rows_v)

            @pl.loop(0, GB, step=16)
            def _(j16):
                dv = pdst_v[pl.ds(b * GB + j16, 16)]
                for j in range(16):
                    d = dv[j]
                    for c in range(8):
                        plsc.addupdate(acc_v.at[d, pl.ds(c * 16, 16)],
                                       rows_v[j16 + j, pl.ds(c * 16, 16)])

            return carry

        lax.fori_loop(0, nb, batch_body, jnp.int32(0))

        @pl.loop(0, RPT, step=16)
        def _(r16):
            hv = hist_v[pl.ds(r16, 16)]
            sc = jnp.where(hv > 0.0, 1.0 / jnp.where(hv > 0.0, hv, 1.0), 0.0)
            for j in range(16):
                s = sc[j]
                for c in range(8):
                    acc_v[r16 + j, pl.ds(c * 16, 16)] = (
                        acc_v[r16 + j, pl.ds(c * 16, 16)] * s)

        pltpu.sync_copy(acc_v, out_hbm.at[pl.ds(lo, RPT)])

    return k(table, dst_flat, src_flat)


def kernel(edge_index, emb, W_hg, b_hg, W_lin, b_lin):
    ei = edge_index.astype(jnp.int32)
    pad = N_NODES + (jnp.arange(NNZ_PAD - NNZ, dtype=jnp.int32) % PAD_ROWS)
    nidx = jnp.concatenate([ei[0], pad])
    hidx = jnp.concatenate([ei[1], pad])
    emb_p = jnp.pad(emb, ((0, PAD_ROWS), (0, 0)))

    x_p = _mm1(emb_p, W_hg)
    oute = _segment_mean(x_p, hidx, nidx)
    out_sc = _segment_mean(oute, nidx, hidx)
    return _final(out_sc, b_hg.reshape(1, C), W_lin, b_lin.reshape(1, C))


# speedup vs baseline: 6.5774x; 1.0547x over previous
import dataclasses
import functools

import jax
import jax.numpy as jnp
from jax import lax
from jax.experimental import pallas as pl
from jax.experimental.pallas import tpu as pltpu
from jax.experimental.pallas import tpu_sc as plsc

N_NODES = 10000
NNZ = 320000
C = 128

NC = 2
NS = 16
NW = NC * NS
PAD_ROWS = 240
R = N_NODES + PAD_ROWS
RPT = R // NW
NNZ_PAD = 327680
CHUNK = 8192
NCHUNK = NNZ_PAD // CHUNK
CAP = 12288
GB = 128
TCB = 2560
TCG = R // TCB

_MESH = plsc.VectorSubcoreMesh(core_axis_name="c", subcore_axis_name="s")
_CP = pltpu.CompilerParams()
if "needs_layout_passes" in pltpu.CompilerParams.__dataclass_fields__:
    _CP = dataclasses.replace(_CP, needs_layout_passes=False)


def _mm1_body(emb_ref, w_ref, o_ref):
    o_ref[...] = lax.dot_general(
        emb_ref[...], w_ref[...], (((1,), (1,)), ((), ())),
        preferred_element_type=jnp.float32)


def _mm1(emb_p, w_hg):
    return pl.pallas_call(
        _mm1_body,
        grid=(TCG,),
        in_specs=[pl.BlockSpec((TCB, C), lambda i: (i, 0)),
                  pl.BlockSpec((C, C), lambda i: (0, 0))],
        out_specs=pl.BlockSpec((TCB, C), lambda i: (i, 0)),
        out_shape=jax.ShapeDtypeStruct((R, C), jnp.float32),
    )(emb_p, w_hg)


def _final_body(p_ref, bhg_ref, wlin_ref, blin_ref, o_ref):
    s = p_ref[...] + bhg_ref[...]
    t = jnp.where(s >= 0.0, s, 0.01 * s)
    o_ref[...] = lax.dot_general(
        t, wlin_ref[...], (((1,), (1,)), ((), ())),
        preferred_element_type=jnp.float32) + blin_ref[...]


def _final(out_sc, b_hg, w_lin, b_lin):
    return pl.pallas_call(
        _final_body,
        grid=(TCG,),
        in_specs=[pl.BlockSpec((TCB, C), lambda i: (i, 0)),
                  pl.BlockSpec((1, C), lambda i: (0, 0)),
                  pl.BlockSpec((C, C), lambda i: (0, 0)),
                  pl.BlockSpec((1, C), lambda i: (0, 0))],
        out_specs=pl.BlockSpec((TCB, C), lambda i: (i, 0)),
        out_shape=jax.ShapeDtypeStruct((N_NODES, C), jnp.float32),
    )(out_sc, b_hg, w_lin, b_lin)


def _segment_mean(table, packed_flat):
    @functools.partial(
        pl.kernel,
        out_type=jax.ShapeDtypeStruct((R, C), jnp.float32),
        mesh=_MESH,
        compiler_params=_CP,
        scratch_types=[
            pltpu.VMEM((2, CHUNK), jnp.int32),
            pltpu.VMEM((CAP,), jnp.int32),
            pltpu.VMEM((CAP,), jnp.int32),
            pltpu.VMEM((RPT,), jnp.float32),
            pltpu.VMEM((2, GB, C), jnp.float32),
            pltpu.VMEM((RPT, C), jnp.float32),
            pltpu.SemaphoreType.DMA,
            pltpu.SemaphoreType.DMA,
            pltpu.SemaphoreType.DMA,
            pltpu.SemaphoreType.DMA,
        ],
    )
    def k(tab_hbm, edg_hbm, out_hbm,
          ech_v, pdst_v, psrc_v, hist_v, rows_v, acc_v,
          csem0, csem1, gsem0, gsem1):
        cid = lax.axis_index("c")
        sid = lax.axis_index("s")
        wid = sid * NC + cid
        lo = wid * RPT

        zero16 = jnp.zeros(16, jnp.float32)
        one16 = jnp.ones(16, jnp.float32)

        @pl.loop(0, RPT)
        def _(r):
            for c in range(8):
                acc_v[r, pl.ds(c * 16, 16)] = zero16

        @pl.loop(0, RPT, step=16)
        def _(i):
            hist_v[pl.ds(i, 16)] = zero16

        @pl.loop(0, CAP, step=16)
        def _(i):
            psrc_v[pl.ds(i, 16)] = jnp.full(16, N_NODES, jnp.int32)
            pdst_v[pl.ds(i, 16)] = jnp.zeros(16, jnp.int32)

        def chunk_dma(ch, buf, sem):
            return pltpu.make_async_copy(
                edg_hbm.at[pl.ds(ch * CHUNK, CHUNK)], ech_v.at[buf], sem)

        def scan_chunk(buf, cnt):
            def vec_body(i, cnt):
                pv = ech_v[buf, pl.ds(i * 16, 16)]
                sv = pv & 16383
                dl = (pv >> 14) - lo
                own = (dl >= 0) & (dl < RPT)
                oi = jnp.where(own, 1, 0)
                pre = plsc.cumsum(oi)
                pos = jnp.minimum(cnt + pre - oi, CAP - 1)
                plsc.store_scatter(psrc_v, [pos], sv, mask=own)
                plsc.store_scatter(pdst_v, [pos], dl, mask=own)
                plsc.addupdate_scatter(hist_v, [dl], one16, mask=own)
                return jnp.minimum(cnt + pre[15], jnp.int32(CAP))

            return lax.fori_loop(0, CHUNK // 16, vec_body, cnt)

        chunk_dma(0, 0, csem0).start()

        def chunk_pair(p, cnt):
            ch0 = 2 * p
            chunk_dma(ch0, 0, csem0).wait()
            chunk_dma(ch0 + 1, 1, csem1).start()
            cnt = scan_chunk(0, cnt)
            chunk_dma(ch0 + 1, 1, csem1).wait()

            @pl.when(p + 1 < NCHUNK // 2)
            def _():
                chunk_dma(ch0 + 2, 0, csem0).start()

            return scan_chunk(1, cnt)

        cnt = lax.fori_loop(0, NCHUNK // 2, chunk_pair, jnp.int32(0))

        nb = (cnt + GB - 1) // GB
        nbp = (nb + 1) // 2

        def gather_dma(b, buf, sem):
            return pltpu.make_async_copy(
                tab_hbm.at[psrc_v.at[pl.ds(b * GB, GB)]], rows_v.at[buf], sem)

        def accumulate(buf, b):
            @pl.loop(0, GB, step=16)
            def _(j16):
                dv = pdst_v[pl.ds(b * GB + j16, 16)]
                for j in range(16):
                    d = dv[j]
                    for c in range(8):
                        plsc.addupdate(acc_v.at[d, pl.ds(c * 16, 16)],
                                       rows_v[buf, j16 + j, pl.ds(c * 16, 16)])

        @pl.when(nb > 0)
        def _():
            gather_dma(0, 0, gsem0).start()

        def gather_pair(p, carry):
            b0 = 2 * p
            gather_dma(b0, 0, gsem0).wait()
            gather_dma(b0 + 1, 1, gsem1).start()
            accumulate(0, b0)
            gather_dma(b0 + 1, 1, gsem1).wait()

            @pl.when(p + 1 < nbp)
            def _():
                gather_dma(b0 + 2, 0, gsem0).start()

            accumulate(1, b0 + 1)
            return carry

        lax.fori_loop(0, nbp, gather_pair, jnp.int32(0))

        @pl.loop(0, RPT, step=16)
        def _(r16):
            hv = hist_v[pl.ds(r16, 16)]
            sc = jnp.where(hv > 0.0, 1.0 / jnp.where(hv > 0.0, hv, 1.0), 0.0)
            for j in range(16):
                s = sc[j]
                for c in range(8):
                    acc_v[r16 + j, pl.ds(c * 16, 16)] = (
                        acc_v[r16 + j, pl.ds(c * 16, 16)] * s)

        pltpu.sync_copy(acc_v, out_hbm.at[pl.ds(lo, RPT)])

    return k(table, packed_flat)


def kernel(edge_index, emb, W_hg, b_hg, W_lin, b_lin):
    ei = edge_index.astype(jnp.int32)
    pad = N_NODES + (jnp.arange(NNZ_PAD - NNZ, dtype=jnp.int32) % PAD_ROWS)
    nidx = jnp.concatenate([ei[0], pad])
    hidx = jnp.concatenate([ei[1], pad])
    packed1 = hidx * 16384 + nidx
    packed2 = nidx * 16384 + hidx
    emb_p = jnp.pad(emb, ((0, PAD_ROWS), (0, 0)))

    x_p = _mm1(emb_p, W_hg)
    oute = _segment_mean(x_p, packed1)
    out_sc = _segment_mean(oute, packed2)
    return _final(out_sc, b_hg.reshape(1, C), W_lin, b_lin.reshape(1, C))


# speedup vs baseline: 7.7455x; 1.1776x over previous
import dataclasses
import functools

import jax
import jax.numpy as jnp
from jax import lax
from jax.experimental import pallas as pl
from jax.experimental.pallas import tpu as pltpu
from jax.experimental.pallas import tpu_sc as plsc

N_NODES = 10000
NNZ = 320000
C = 128

NC = 2
NS = 16
NW = NC * NS
PAD_ROWS = 240
R = N_NODES + PAD_ROWS
RPT = R // NW
NNZ_PAD = 327680
CHUNK = 8192
NCHUNK = NNZ_PAD // CHUNK
CAP = 12288
GB = 128
TCB = 2560
TCG = R // TCB

_MESH = plsc.VectorSubcoreMesh(core_axis_name="c", subcore_axis_name="s")
_CP = pltpu.CompilerParams()
if "needs_layout_passes" in pltpu.CompilerParams.__dataclass_fields__:
    _CP = dataclasses.replace(_CP, needs_layout_passes=False)


def _mm1_body(emb_ref, w_ref, o_ref):
    o_ref[...] = lax.dot_general(
        emb_ref[...], w_ref[...], (((1,), (1,)), ((), ())),
        preferred_element_type=jnp.float32)


def _mm1(emb_p, w_hg):
    return pl.pallas_call(
        _mm1_body,
        grid=(TCG,),
        in_specs=[pl.BlockSpec((TCB, C), lambda i: (i, 0)),
                  pl.BlockSpec((C, C), lambda i: (0, 0))],
        out_specs=pl.BlockSpec((TCB, C), lambda i: (i, 0)),
        out_shape=jax.ShapeDtypeStruct((R, C), jnp.float32),
    )(emb_p, w_hg)


def _final_body(p_ref, bhg_ref, wlin_ref, blin_ref, o_ref):
    s = p_ref[...] + bhg_ref[...]
    t = jnp.where(s >= 0.0, s, 0.01 * s)
    o_ref[...] = lax.dot_general(
        t, wlin_ref[...], (((1,), (1,)), ((), ())),
        preferred_element_type=jnp.float32) + blin_ref[...]


def _final(out_sc, b_hg, w_lin, b_lin):
    return pl.pallas_call(
        _final_body,
        grid=(TCG,),
        in_specs=[pl.BlockSpec((TCB, C), lambda i: (i, 0)),
                  pl.BlockSpec((1, C), lambda i: (0, 0)),
                  pl.BlockSpec((C, C), lambda i: (0, 0)),
                  pl.BlockSpec((1, C), lambda i: (0, 0))],
        out_specs=pl.BlockSpec((TCB, C), lambda i: (i, 0)),
        out_shape=jax.ShapeDtypeStruct((N_NODES, C), jnp.float32),
    )(out_sc, b_hg, w_lin, b_lin)


def _segment_mean(table, packed_flat):
    @functools.partial(
        pl.kernel,
        out_type=jax.ShapeDtypeStruct((R, C), jnp.float32),
        mesh=_MESH,
        compiler_params=_CP,
        scratch_types=[
            pltpu.VMEM((2, CHUNK), jnp.int32),
            pltpu.VMEM((CAP,), jnp.int32),
            pltpu.VMEM((CAP,), jnp.int32),
            pltpu.VMEM((RPT,), jnp.float32),
            pltpu.VMEM((2, GB, C), jnp.float32),
            pltpu.VMEM((RPT, C), jnp.float32),
            pltpu.SemaphoreType.DMA,
            pltpu.SemaphoreType.DMA,
            pltpu.SemaphoreType.DMA,
            pltpu.SemaphoreType.DMA,
        ],
    )
    def k(tab_hbm, edg_hbm, out_hbm,
          ech_v, pdst_v, psrc_v, hist_v, rows_v, acc_v,
          csem0, csem1, gsem0, gsem1):
        cid = lax.axis_index("c")
        sid = lax.axis_index("s")
        wid = sid * NC + cid
        lo = wid * RPT

        zero16 = jnp.zeros(16, jnp.float32)
        one16 = jnp.ones(16, jnp.float32)

        @pl.loop(0, RPT)
        def _(r):
            for c in range(8):
                acc_v[r, pl.ds(c * 16, 16)] = zero16

        @pl.loop(0, RPT, step=16)
        def _(i):
            hist_v[pl.ds(i, 16)] = zero16

        @pl.loop(0, CAP, step=16)
        def _(i):
            psrc_v[pl.ds(i, 16)] = jnp.full(16, N_NODES, jnp.int32)
            pdst_v[pl.ds(i, 16)] = jnp.zeros(16, jnp.int32)

        def chunk_dma(ch, buf, sem):
            return pltpu.make_async_copy(
                edg_hbm.at[pl.ds(ch * CHUNK, CHUNK)], ech_v.at[buf], sem)

        def scan_chunk(buf, cnt):
            def vec_body(i, cnt):
                pv = ech_v[buf, pl.ds(i * 16, 16)]
                sv = pv & 16383
                dl = (pv >> 14) - lo
                own = (dl >= 0) & (dl < RPT)
                plsc.store_compressed(psrc_v.at[pl.ds(cnt, 16)], sv, mask=own)
                plsc.store_compressed(pdst_v.at[pl.ds(cnt, 16)], dl, mask=own)
                pc = plsc.all_reduce_population_count(own)
                return jnp.minimum(cnt + pc[0], jnp.int32(CAP - 16))

            return lax.fori_loop(0, CHUNK // 16, vec_body, cnt)

        chunk_dma(0, 0, csem0).start()

        def chunk_pair(p, cnt):
            ch0 = 2 * p
            chunk_dma(ch0, 0, csem0).wait()
            chunk_dma(ch0 + 1, 1, csem1).start()
            cnt = scan_chunk(0, cnt)
            chunk_dma(ch0 + 1, 1, csem1).wait()

            @pl.when(p + 1 < NCHUNK // 2)
            def _():
                chunk_dma(ch0 + 2, 0, csem0).start()

            return scan_chunk(1, cnt)

        cnt = lax.fori_loop(0, NCHUNK // 2, chunk_pair, jnp.int32(0))

        iota16 = lax.iota(jnp.int32, 16)

        def hist_body(i, carry):
            dv = pdst_v[pl.ds(i * 16, 16)]
            valid = (i * 16 + iota16) < cnt
            plsc.addupdate_scatter(hist_v, [dv], one16, mask=valid)
            return carry

        lax.fori_loop(0, (cnt + 15) // 16, hist_body, jnp.int32(0))

        nb = (cnt + GB - 1) // GB
        nbp = (nb + 1) // 2

        def gather_dma(b, buf, sem):
            return pltpu.make_async_copy(
                tab_hbm.at[psrc_v.at[pl.ds(b * GB, GB)]], rows_v.at[buf], sem)

        def accumulate(buf, b):
            @pl.loop(0, GB, step=16)
            def _(j16):
                dv = pdst_v[pl.ds(b * GB + j16, 16)]
                for j in range(16):
                    d = dv[j]
                    for c in range(8):
                        plsc.addupdate(acc_v.at[d, pl.ds(c * 16, 16)],
                                       rows_v[buf, j16 + j, pl.ds(c * 16, 16)])

        @pl.when(nb > 0)
        def _():
            gather_dma(0, 0, gsem0).start()

        def gather_pair(p, carry):
            b0 = 2 * p
            gather_dma(b0, 0, gsem0).wait()
            gather_dma(b0 + 1, 1, gsem1).start()
            accumulate(0, b0)
            gather_dma(b0 + 1, 1, gsem1).wait()

            @pl.when(p + 1 < nbp)
            def _():
                gather_dma(b0 + 2, 0, gsem0).start()

            accumulate(1, b0 + 1)
            return carry

        lax.fori_loop(0, nbp, gather_pair, jnp.int32(0))

        @pl.loop(0, RPT, step=16)
        def _(r16):
            hv = hist_v[pl.ds(r16, 16)]
            sc = jnp.where(hv > 0.0, 1.0 / jnp.where(hv > 0.0, hv, 1.0), 0.0)
            for j in range(16):
                s = sc[j]
                for c in range(8):
                    acc_v[r16 + j, pl.ds(c * 16, 16)] = (
                        acc_v[r16 + j, pl.ds(c * 16, 16)] * s)

        pltpu.sync_copy(acc_v, out_hbm.at[pl.ds(lo, RPT)])

    return k(table, packed_flat)


def kernel(edge_index, emb, W_hg, b_hg, W_lin, b_lin):
    ei = edge_index.astype(jnp.int32)
    pad = N_NODES + (jnp.arange(NNZ_PAD - NNZ, dtype=jnp.int32) % PAD_ROWS)
    nidx = jnp.concatenate([ei[0], pad])
    hidx = jnp.concatenate([ei[1], pad])
    packed1 = hidx * 16384 + nidx
    packed2 = nidx * 16384 + hidx
    emb_p = jnp.pad(emb, ((0, PAD_ROWS), (0, 0)))

    x_p = _mm1(emb_p, W_hg)
    oute = _segment_mean(x_p, packed1)
    out_sc = _segment_mean(oute, packed2)
    return _final(out_sc, b_hg.reshape(1, C), W_lin, b_lin.reshape(1, C))
